# baseline (speedup 1.0000x reference)
import functools

import jax
import jax.numpy as jnp
from jax import lax
from jax.experimental import pallas as pl
from jax.experimental.pallas import tpu as pltpu
from jax.experimental.pallas import tpu_sc as plsc

NUM_RELS = 100000
EMB_DIM = 128
BATCH = 16384
HALF = BATCH // 2
_ENC_SCALE = 65536.0 / (2.0 * 3.141592653589793)
_DEC_SCALE = (2.0 * 3.141592653589793) / 65536.0


@functools.lru_cache(maxsize=None)
def _make_sc_gather_pack(V, D, B):
    NC, NS = 2, 16
    NW = NC * NS
    half = B // 2
    p_per_w = half // NW
    mesh = plsc.VectorSubcoreMesh(core_axis_name="c", subcore_axis_name="s")

    ch = 128
    nch = p_per_w // ch

    @functools.partial(
        pl.kernel,
        mesh=mesh,
        out_type=jax.ShapeDtypeStruct((half, D), jnp.int32),
        scratch_types=[
            pltpu.VMEM((p_per_w,), jnp.int32),
            pltpu.VMEM((p_per_w,), jnp.int32),
            pltpu.VMEM((p_per_w, D), jnp.float32),
            pltpu.VMEM((p_per_w, D), jnp.float32),
            pltpu.VMEM((p_per_w, D), jnp.int32),
            pltpu.SemaphoreType.DMA,
            pltpu.SemaphoreType.DMA,
        ],
    )
    def gather_k(table_hbm, idx_hbm, out_hbm, idx_lo, idx_hi, rows_lo, rows_hi,
                 packed_v, gsem, wsem):
        wid = lax.axis_index("s") * NC + lax.axis_index("c")
        base = wid * p_per_w
        pltpu.sync_copy(idx_hbm.at[pl.ds(base, p_per_w)], idx_lo)
        pltpu.sync_copy(idx_hbm.at[pl.ds(half + base, p_per_w)], idx_hi)
        gathers = []
        for j in range(nch):
            sl = pl.ds(j * ch, ch)
            g_lo = pltpu.async_copy(table_hbm.at[idx_lo.at[sl]], rows_lo.at[sl],
                                    gsem)
            g_hi = pltpu.async_copy(table_hbm.at[idx_hi.at[sl]], rows_hi.at[sl],
                                    gsem)
            gathers.append((g_lo, g_hi))

        def pack_row(p, _):
            for k in range(D // 16):
                sl = pl.ds(k * 16, 16)
                qa = lax.convert_element_type(
                    rows_lo[p, sl] * _ENC_SCALE, jnp.int32) & 65535
                qb = lax.convert_element_type(
                    rows_hi[p, sl] * _ENC_SCALE, jnp.int32)
                packed_v[p, sl] = qa | (qb << 16)
            return ()

        writes = []
        for j in range(nch):
            g_lo, g_hi = gathers[j]
            g_lo.wait()
            g_hi.wait()
            lax.fori_loop(j * ch, (j + 1) * ch, pack_row, ())
            sl = pl.ds(j * ch, ch)
            writes.append(
                pltpu.async_copy(packed_v.at[sl],
                                 out_hbm.at[pl.ds(base + j * ch, ch)], wsem))
        for w in writes:
            w.wait()

    return gather_k


_SIN_COEFFS = (0.9999998622, -0.1666660773, 8.332732438e-3,
               -1.981669233e-4, 2.708326132e-6, -2.069597016e-8)
_COS_COEFFS = (0.9999999739, -0.4999998513, 4.166646236e-2,
               -1.38877318e-3, 2.476905337e-5, -2.70754507e-7,
               1.724375218e-9)


def _poly(y, coeffs):
    acc = coeffs[-1]
    for cf in coeffs[-2::-1]:
        acc = cf + y * acc
    return acc


def _half_score(head, tail, ph):
    u = ph - jnp.float32(jnp.pi)
    y = u * u
    su = u * _poly(y, _SIN_COEFFS)
    cu = _poly(y, _COS_COEFFS)
    hr = head[:, :EMB_DIM]
    hi = head[:, EMB_DIM:]
    re = hi * su - hr * cu - tail[:, :EMB_DIM]
    im = hr * su + hi * cu + tail[:, EMB_DIM:]
    dist = jnp.sqrt(re * re + im * im)
    blk = dist.shape[0]
    parts = []
    for j in range(blk // EMB_DIM):
        chunk = dist[j * EMB_DIM:(j + 1) * EMB_DIM, :]
        parts.append(jnp.sum(chunk.T, axis=0))
    return -jnp.concatenate(parts, axis=0)


def _score_body(head_lo_ref, tail_lo_ref, head_hi_ref, tail_hi_ref, pk_ref,
                out_lo_ref, out_hi_ref):
    x = pk_ref[...]
    ph_lo = lax.convert_element_type(x & 65535, jnp.float32) * jnp.float32(
        _DEC_SCALE)
    ph_hi = lax.convert_element_type(
        lax.shift_right_logical(x, 16), jnp.float32) * jnp.float32(_DEC_SCALE)
    out_lo_ref[...] = _half_score(head_lo_ref[...], tail_lo_ref[...], ph_lo)
    out_hi_ref[...] = _half_score(head_hi_ref[...], tail_hi_ref[...], ph_hi)


def _tc_score(head_emb, tail_emb, packed):
    blk = 2048
    nblk = HALF // blk
    emb_spec_lo = pl.BlockSpec((blk, 2 * EMB_DIM), lambda i: (i, 0))
    emb_spec_hi = pl.BlockSpec((blk, 2 * EMB_DIM), lambda i, n=nblk: (i + n, 0))
    out_lo, out_hi = pl.pallas_call(
        _score_body,
        grid=(nblk,),
        in_specs=[
            emb_spec_lo,
            emb_spec_lo,
            emb_spec_hi,
            emb_spec_hi,
            pl.BlockSpec((blk, EMB_DIM), lambda i: (i, 0)),
        ],
        out_specs=[
            pl.BlockSpec((blk,), lambda i: (i,)),
            pl.BlockSpec((blk,), lambda i: (i,)),
        ],
        out_shape=[
            jax.ShapeDtypeStruct((HALF,), jnp.float32),
            jax.ShapeDtypeStruct((HALF,), jnp.float32),
        ],
    )(head_emb, tail_emb, head_emb, tail_emb, packed)
    return jnp.concatenate([out_lo, out_hi], axis=0)


def kernel(head_emb, tail_emb, rel_table, rel_idx):
    packed = _make_sc_gather_pack(NUM_RELS, EMB_DIM, BATCH)(
        rel_table, rel_idx.astype(jnp.int32))
    return _tc_score(head_emb, tail_emb, packed)
